# Initial kernel scaffold; baseline (speedup 1.0000x reference)
import jax
import jax.numpy as jnp
from jax import lax
from jax.experimental import pallas as pl
from jax.experimental.pallas import tpu as pltpu
from jax.experimental.pallas import tpu_sc as plsc

VOCAB = 100000
EMBED_DIM = 64
BATCH = 4096
HIST = 50

NC = 2
NS = 16
NW = NC * NS

TOTAL = BATCH * HIST
PER_W = TOTAL // NW
CH = 128
NCH = PER_W // CH


def _glove_body(words_h, ctx_h, wemb_h, wbias_h, cemb_h, cbias_h,
                out_we, out_wb, out_ce, out_cb,
                idx_w, idx_c, wbuf, cbuf, wbias_v, cbias_v, sem):
    wid = lax.axis_index("s") * NC + lax.axis_index("c")
    base = wid * PER_W

    pltpu.sync_copy(words_h.at[wid], idx_w)
    pltpu.sync_copy(ctx_h.at[wid], idx_c)

    def step(j, carry):
        row0 = base + j * CH
        pltpu.async_copy(wemb_h.at[idx_w.at[j]], wbuf, sem).wait()
        pltpu.sync_copy(wbuf, out_we.at[pl.ds(row0, CH)])
        pltpu.async_copy(cemb_h.at[idx_c.at[j]], cbuf, sem).wait()
        pltpu.sync_copy(cbuf, out_ce.at[pl.ds(row0, CH)])
        pltpu.async_copy(wbias_h.at[idx_w.at[j]],
                         wbias_v.at[pl.ds(j * CH, CH)], sem).wait()
        pltpu.async_copy(cbias_h.at[idx_c.at[j]],
                         cbias_v.at[pl.ds(j * CH, CH)], sem).wait()
        return carry

    lax.fori_loop(0, NCH, step, 0)

    pltpu.sync_copy(wbias_v, out_wb.at[pl.ds(base, PER_W)])
    pltpu.sync_copy(cbias_v, out_cb.at[pl.ds(base, PER_W)])


@jax.jit
def _glove_sc(words3, ctx3, w_embeddings, w_biases, c_embeddings, c_biases):
    mesh = plsc.VectorSubcoreMesh(core_axis_name="c", subcore_axis_name="s",
                                  num_cores=NC, num_subcores=NS)
    f32 = jnp.float32
    run = pl.kernel(
        _glove_body,
        out_type=(
            jax.ShapeDtypeStruct((TOTAL, EMBED_DIM), f32),
            jax.ShapeDtypeStruct((TOTAL, 1), f32),
            jax.ShapeDtypeStruct((TOTAL, EMBED_DIM), f32),
            jax.ShapeDtypeStruct((TOTAL, 1), f32),
        ),
        mesh=mesh,
        scratch_types=[
            pltpu.VMEM((NCH, CH), jnp.int32),
            pltpu.VMEM((NCH, CH), jnp.int32),
            pltpu.VMEM((CH, EMBED_DIM), f32),
            pltpu.VMEM((CH, EMBED_DIM), f32),
            pltpu.VMEM((PER_W, 1), f32),
            pltpu.VMEM((PER_W, 1), f32),
            pltpu.SemaphoreType.DMA,
        ],
    )
    return run(words3, ctx3, w_embeddings, w_biases, c_embeddings, c_biases)


def kernel(words, contexts, w_embeddings, w_biases, c_embeddings, c_biases):
    words3 = words.astype(jnp.int32).reshape(NW, NCH, CH)
    ctx3 = contexts.astype(jnp.int32).reshape(NW, NCH, CH)
    we, wb, ce, cb = _glove_sc(words3, ctx3, w_embeddings, w_biases,
                               c_embeddings, c_biases)
    return (
        we.reshape(BATCH, HIST, EMBED_DIM),
        wb.reshape(BATCH, HIST, 1),
        ce.reshape(BATCH, HIST, EMBED_DIM),
        cb.reshape(BATCH, HIST, 1),
    )


# speedup vs baseline: 8.9454x; 8.9454x over previous
import jax
import jax.numpy as jnp
from jax import lax
from jax.experimental import pallas as pl
from jax.experimental.pallas import tpu as pltpu
from jax.experimental.pallas import tpu_sc as plsc

VOCAB = 100000
EMBED_DIM = 64
BATCH = 4096
HIST = 50

NC = 2
NS = 16
NW = NC * NS

TOTAL = BATCH * HIST
PER_W = TOTAL // NW
CH = 128
NCH = PER_W // CH


def _glove_body(words_h, ctx_h, wemb_h, wbias_h, cemb_h, cbias_h,
                out_we, out_wb, out_ce, out_cb,
                idx_w, idx_c, wbuf, cbuf, wbias_v, cbias_v, sem):
    wid = lax.axis_index("s") * NC + lax.axis_index("c")
    base = wid * PER_W

    pltpu.sync_copy(words_h.at[wid], idx_w)
    pltpu.sync_copy(ctx_h.at[wid], idx_c)

    def step(j, carry):
        row0 = base + j * CH
        pltpu.async_copy(wemb_h.at[idx_w.at[j]], wbuf, sem).wait()
        pltpu.sync_copy(wbuf, out_we.at[pl.ds(row0, CH)])
        pltpu.async_copy(cemb_h.at[idx_c.at[j]], cbuf, sem).wait()
        pltpu.sync_copy(cbuf, out_ce.at[pl.ds(row0, CH)])
        pltpu.async_copy(wbias_h.at[idx_w.at[j]],
                         wbias_v.at[pl.ds(j * CH, CH)], sem).wait()
        pltpu.async_copy(cbias_h.at[idx_c.at[j]],
                         cbias_v.at[pl.ds(j * CH, CH)], sem).wait()
        return carry

    lax.fori_loop(0, NCH, step, 0)

    pltpu.sync_copy(wbias_v, out_wb.at[pl.ds(base, PER_W)])
    pltpu.sync_copy(cbias_v, out_cb.at[pl.ds(base, PER_W)])


@jax.jit
def _glove_sc(words3, ctx3, w_embeddings, w_biases, c_embeddings, c_biases):
    mesh = plsc.VectorSubcoreMesh(core_axis_name="c", subcore_axis_name="s",
                                  num_cores=NC, num_subcores=NS)
    f32 = jnp.float32
    run = pl.kernel(
        _glove_body,
        out_type=(
            jax.ShapeDtypeStruct((TOTAL, EMBED_DIM), f32),
            jax.ShapeDtypeStruct((TOTAL,), f32),
            jax.ShapeDtypeStruct((TOTAL, EMBED_DIM), f32),
            jax.ShapeDtypeStruct((TOTAL,), f32),
        ),
        mesh=mesh,
        compiler_params=pltpu.CompilerParams(use_tc_tiling_on_sc=False),
        scratch_types=[
            pltpu.VMEM((NCH, CH), jnp.int32),
            pltpu.VMEM((NCH, CH), jnp.int32),
            pltpu.VMEM((CH, EMBED_DIM), f32),
            pltpu.VMEM((CH, EMBED_DIM), f32),
            pltpu.VMEM((PER_W,), f32),
            pltpu.VMEM((PER_W,), f32),
            pltpu.SemaphoreType.DMA,
        ],
    )
    return run(words3, ctx3, w_embeddings, w_biases, c_embeddings, c_biases)


def kernel(words, contexts, w_embeddings, w_biases, c_embeddings, c_biases):
    words3 = words.astype(jnp.int32).reshape(NW, NCH, CH)
    ctx3 = contexts.astype(jnp.int32).reshape(NW, NCH, CH)
    we, wb, ce, cb = _glove_sc(words3, ctx3,
                               w_embeddings, w_biases.reshape(VOCAB),
                               c_embeddings, c_biases.reshape(VOCAB))
    return (
        we.reshape(BATCH, HIST, EMBED_DIM),
        wb.reshape(BATCH, HIST, 1),
        ce.reshape(BATCH, HIST, EMBED_DIM),
        cb.reshape(BATCH, HIST, 1),
    )


# speedup vs baseline: 11.8311x; 1.3226x over previous
import jax
import jax.numpy as jnp
from jax import lax
from jax.experimental import pallas as pl
from jax.experimental.pallas import tpu as pltpu
from jax.experimental.pallas import tpu_sc as plsc

VOCAB = 100000
EMBED_DIM = 64
BATCH = 4096
HIST = 50

NC = 2
NS = 16
NW = NC * NS

TOTAL = BATCH * HIST
PER_W = TOTAL // NW
CH = 128
NCH = PER_W // CH
NSLOT = 4
LOOKAHEAD = 2


def _glove_body(words_h, ctx_h, wemb_h, wbias_h, cemb_h, cbias_h,
                out_we, out_wb, out_ce, out_cb,
                idx_w, idx_c, wbuf, cbuf, wbias_v, cbias_v,
                gsem_w, gsem_c, wsem_w, wsem_c, bsem):
    wid = lax.axis_index("s") * NC + lax.axis_index("c")
    base = wid * PER_W

    pltpu.sync_copy(words_h.at[wid], idx_w)
    pltpu.sync_copy(ctx_h.at[wid], idx_c)

    for j in range(LOOKAHEAD):
        pltpu.async_copy(wemb_h.at[idx_w.at[j]], wbuf.at[j], gsem_w.at[j])
        pltpu.async_copy(cemb_h.at[idx_c.at[j]], cbuf.at[j], gsem_c.at[j])

    def step(j, carry):
        s = lax.rem(j, NSLOT)
        row0 = base + j * CH
        pltpu.make_async_copy(wemb_h.at[idx_w.at[j]], wbuf.at[s],
                              gsem_w.at[s]).wait()
        pltpu.async_copy(wbuf.at[s], out_we.at[pl.ds(row0, CH)], wsem_w.at[s])
        pltpu.make_async_copy(cemb_h.at[idx_c.at[j]], cbuf.at[s],
                              gsem_c.at[s]).wait()
        pltpu.async_copy(cbuf.at[s], out_ce.at[pl.ds(row0, CH)], wsem_c.at[s])
        pltpu.async_copy(wbias_h.at[idx_w.at[j]],
                         wbias_v.at[pl.ds(j * CH, CH)], bsem)
        pltpu.async_copy(cbias_h.at[idx_c.at[j]],
                         cbias_v.at[pl.ds(j * CH, CH)], bsem)
        jn = j + LOOKAHEAD
        sn = lax.rem(jn, NSLOT)

        @pl.when(jn < NCH)
        def _():
            @pl.when(jn >= NSLOT)
            def _():
                rprev = base + (jn - NSLOT) * CH
                pltpu.make_async_copy(wbuf.at[sn],
                                      out_we.at[pl.ds(rprev, CH)],
                                      wsem_w.at[sn]).wait()
                pltpu.make_async_copy(cbuf.at[sn],
                                      out_ce.at[pl.ds(rprev, CH)],
                                      wsem_c.at[sn]).wait()
            pltpu.async_copy(wemb_h.at[idx_w.at[jn]], wbuf.at[sn],
                             gsem_w.at[sn])
            pltpu.async_copy(cemb_h.at[idx_c.at[jn]], cbuf.at[sn],
                             gsem_c.at[sn])
        return carry

    lax.fori_loop(0, NCH, step, 0)

    for s in range(NSLOT):
        rlast = base + (NCH - NSLOT + s) * CH
        pltpu.make_async_copy(wbuf.at[s], out_we.at[pl.ds(rlast, CH)],
                              wsem_w.at[s]).wait()
        pltpu.make_async_copy(cbuf.at[s], out_ce.at[pl.ds(rlast, CH)],
                              wsem_c.at[s]).wait()

    pltpu.make_async_copy(wbias_h.at[pl.ds(0, PER_W)], wbias_v, bsem).wait()
    pltpu.make_async_copy(cbias_h.at[pl.ds(0, PER_W)], cbias_v, bsem).wait()
    pltpu.sync_copy(wbias_v, out_wb.at[pl.ds(base, PER_W)])
    pltpu.sync_copy(cbias_v, out_cb.at[pl.ds(base, PER_W)])


@jax.jit
def _glove_sc(words3, ctx3, w_embeddings, w_biases, c_embeddings, c_biases):
    mesh = plsc.VectorSubcoreMesh(core_axis_name="c", subcore_axis_name="s",
                                  num_cores=NC, num_subcores=NS)
    f32 = jnp.float32
    run = pl.kernel(
        _glove_body,
        out_type=(
            jax.ShapeDtypeStruct((TOTAL, EMBED_DIM), f32),
            jax.ShapeDtypeStruct((TOTAL,), f32),
            jax.ShapeDtypeStruct((TOTAL, EMBED_DIM), f32),
            jax.ShapeDtypeStruct((TOTAL,), f32),
        ),
        mesh=mesh,
        compiler_params=pltpu.CompilerParams(use_tc_tiling_on_sc=False),
        scratch_types=[
            pltpu.VMEM((NCH, CH), jnp.int32),
            pltpu.VMEM((NCH, CH), jnp.int32),
            pltpu.VMEM((NSLOT, CH, EMBED_DIM), f32),
            pltpu.VMEM((NSLOT, CH, EMBED_DIM), f32),
            pltpu.VMEM((PER_W,), f32),
            pltpu.VMEM((PER_W,), f32),
            pltpu.SemaphoreType.DMA((NSLOT,)),
            pltpu.SemaphoreType.DMA((NSLOT,)),
            pltpu.SemaphoreType.DMA((NSLOT,)),
            pltpu.SemaphoreType.DMA((NSLOT,)),
            pltpu.SemaphoreType.DMA,
        ],
    )
    return run(words3, ctx3, w_embeddings, w_biases, c_embeddings, c_biases)


def kernel(words, contexts, w_embeddings, w_biases, c_embeddings, c_biases):
    words3 = words.astype(jnp.int32).reshape(NW, NCH, CH)
    ctx3 = contexts.astype(jnp.int32).reshape(NW, NCH, CH)
    we, wb, ce, cb = _glove_sc(words3, ctx3,
                               w_embeddings, w_biases.reshape(VOCAB),
                               c_embeddings, c_biases.reshape(VOCAB))
    return (
        we.reshape(BATCH, HIST, EMBED_DIM),
        wb.reshape(BATCH, HIST, 1),
        ce.reshape(BATCH, HIST, EMBED_DIM),
        cb.reshape(BATCH, HIST, 1),
    )


# speedup vs baseline: 12.2888x; 1.0387x over previous
import jax
import jax.numpy as jnp
from jax import lax
from jax.experimental import pallas as pl
from jax.experimental.pallas import tpu as pltpu
from jax.experimental.pallas import tpu_sc as plsc

VOCAB = 100000
EMBED_DIM = 64
BATCH = 4096
HIST = 50

NC = 2
NS = 16
NW = NC * NS

CH = BATCH // NW
NSLOT = 4
LOOKAHEAD = 2


def _glove_body(words_h, ctx_h, wemb_h, wbias_h, cemb_h, cbias_h,
                out_we, out_wb, out_ce, out_cb,
                idx_w, idx_c, wbuf, cbuf, wbias_v, cbias_v,
                gsem_w, gsem_c, wsem_w, wsem_c, bsem):
    wid = lax.axis_index("s") * NC + lax.axis_index("c")
    b0 = wid * CH

    pltpu.sync_copy(words_h.at[:, pl.ds(b0, CH)], idx_w)
    pltpu.sync_copy(ctx_h.at[:, pl.ds(b0, CH)], idx_c)

    for h in range(LOOKAHEAD):
        pltpu.async_copy(wemb_h.at[idx_w.at[h]], wbuf.at[h], gsem_w.at[h])
        pltpu.async_copy(cemb_h.at[idx_c.at[h]], cbuf.at[h], gsem_c.at[h])

    def step(h, carry):
        s = lax.rem(h, NSLOT)
        pltpu.make_async_copy(wemb_h.at[idx_w.at[h]], wbuf.at[s],
                              gsem_w.at[s]).wait()
        pltpu.async_copy(wbuf.at[s], out_we.at[h, pl.ds(b0, CH)], wsem_w.at[s])
        pltpu.make_async_copy(cemb_h.at[idx_c.at[h]], cbuf.at[s],
                              gsem_c.at[s]).wait()
        pltpu.async_copy(cbuf.at[s], out_ce.at[h, pl.ds(b0, CH)], wsem_c.at[s])
        pltpu.async_copy(wbias_h.at[idx_w.at[h]], wbias_v.at[h], bsem)
        pltpu.async_copy(cbias_h.at[idx_c.at[h]], cbias_v.at[h], bsem)
        hn = h + LOOKAHEAD
        sn = lax.rem(hn, NSLOT)

        @pl.when(hn < HIST)
        def _():
            @pl.when(hn >= NSLOT)
            def _():
                hprev = hn - NSLOT
                pltpu.make_async_copy(wbuf.at[sn],
                                      out_we.at[hprev, pl.ds(b0, CH)],
                                      wsem_w.at[sn]).wait()
                pltpu.make_async_copy(cbuf.at[sn],
                                      out_ce.at[hprev, pl.ds(b0, CH)],
                                      wsem_c.at[sn]).wait()
            pltpu.async_copy(wemb_h.at[idx_w.at[hn]], wbuf.at[sn],
                             gsem_w.at[sn])
            pltpu.async_copy(cemb_h.at[idx_c.at[hn]], cbuf.at[sn],
                             gsem_c.at[sn])
        return carry

    lax.fori_loop(0, HIST, step, 0)

    for s in range(NSLOT):
        hlast = HIST - NSLOT + s
        pltpu.make_async_copy(wbuf.at[s], out_we.at[hlast, pl.ds(b0, CH)],
                              wsem_w.at[s]).wait()
        pltpu.make_async_copy(cbuf.at[s], out_ce.at[hlast, pl.ds(b0, CH)],
                              wsem_c.at[s]).wait()

    pltpu.make_async_copy(wbias_h.at[pl.ds(0, HIST * CH)],
                          wbias_v, bsem).wait()
    pltpu.make_async_copy(cbias_h.at[pl.ds(0, HIST * CH)],
                          cbias_v, bsem).wait()
    pltpu.sync_copy(wbias_v, out_wb.at[:, pl.ds(b0, CH)])
    pltpu.sync_copy(cbias_v, out_cb.at[:, pl.ds(b0, CH)])


@jax.jit
def _glove_sc(wordsT, ctxT, w_embeddings, w_biases, c_embeddings, c_biases):
    mesh = plsc.VectorSubcoreMesh(core_axis_name="c", subcore_axis_name="s",
                                  num_cores=NC, num_subcores=NS)
    f32 = jnp.float32
    run = pl.kernel(
        _glove_body,
        out_type=(
            jax.ShapeDtypeStruct((HIST, BATCH, EMBED_DIM), f32),
            jax.ShapeDtypeStruct((HIST, BATCH), f32),
            jax.ShapeDtypeStruct((HIST, BATCH, EMBED_DIM), f32),
            jax.ShapeDtypeStruct((HIST, BATCH), f32),
        ),
        mesh=mesh,
        compiler_params=pltpu.CompilerParams(use_tc_tiling_on_sc=False),
        scratch_types=[
            pltpu.VMEM((HIST, CH), jnp.int32),
            pltpu.VMEM((HIST, CH), jnp.int32),
            pltpu.VMEM((NSLOT, CH, EMBED_DIM), f32),
            pltpu.VMEM((NSLOT, CH, EMBED_DIM), f32),
            pltpu.VMEM((HIST, CH), f32),
            pltpu.VMEM((HIST, CH), f32),
            pltpu.SemaphoreType.DMA((NSLOT,)),
            pltpu.SemaphoreType.DMA((NSLOT,)),
            pltpu.SemaphoreType.DMA((NSLOT,)),
            pltpu.SemaphoreType.DMA((NSLOT,)),
            pltpu.SemaphoreType.DMA,
        ],
    )
    return run(wordsT, ctxT, w_embeddings, w_biases, c_embeddings, c_biases)


def kernel(words, contexts, w_embeddings, w_biases, c_embeddings, c_biases):
    wordsT = words.astype(jnp.int32).T
    ctxT = contexts.astype(jnp.int32).T
    we, wb, ce, cb = _glove_sc(wordsT, ctxT,
                               w_embeddings, w_biases.reshape(VOCAB),
                               c_embeddings, c_biases.reshape(VOCAB))
    return (
        jnp.transpose(we, (1, 0, 2)),
        wb.T.reshape(BATCH, HIST, 1),
        jnp.transpose(ce, (1, 0, 2)),
        cb.T.reshape(BATCH, HIST, 1),
    )
